# Initial kernel scaffold; baseline (speedup 1.0000x reference)
import jax
import jax.numpy as jnp
from jax.experimental import pallas as pl


def kernel(z, emb):
    raise NotImplementedError("write your pallas kernel here")


# speedup vs baseline: 1.8928x; 1.8928x over previous
import jax
import jax.numpy as jnp
from jax.experimental import pallas as pl
from jax.experimental.pallas import tpu as pltpu

_NE = 1024
_ED = 64
_CC = 0.25
_EPS = 1e-12


def _vq_body(z_ref, emb_ref, ehi_ref, elo_ref, q_ref, idx_ref, loss_ref, acc_ref):
    b = pl.program_id(0)
    nb = pl.num_programs(0)
    zb = z_ref[0]
    emb = emb_ref[...]

    en = emb / jnp.clip(jnp.sqrt(jnp.sum(emb * emb, axis=1, keepdims=True)), _EPS, None)
    c = jnp.sum(en * en, axis=1, keepdims=True)

    ssq = jnp.sum(zb * zb, axis=0, keepdims=True)
    fn = zb / jnp.clip(jnp.sqrt(ssq), _EPS, None)

    s = jax.lax.dot_general(
        en.astype(jnp.bfloat16), fn.astype(jnp.bfloat16),
        (((1,), (0,)), ((), ())), preferred_element_type=jnp.float32)
    d = c - 2.0 * s

    dmin = jnp.min(d, axis=0, keepdims=True)
    jids = jax.lax.broadcasted_iota(jnp.int32, (_NE, _NE), 0)
    idx = jnp.min(jnp.where(d == dmin, jids, _NE), axis=0, keepdims=True)
    idx_ref[0] = idx

    oh = (jids == idx).astype(jnp.bfloat16)
    q = (jax.lax.dot_general(ehi_ref[...], oh, (((1,), (0,)), ((), ())),
                             preferred_element_type=jnp.float32)
         + jax.lax.dot_general(elo_ref[...], oh, (((1,), (0,)), ((), ())),
                               preferred_element_type=jnp.float32))
    q_ref[0] = q

    diff = q - zb
    part = jnp.sum(diff * diff).reshape(1, 1)

    @pl.when(b == 0)
    def _init():
        acc_ref[...] = jnp.zeros_like(acc_ref)

    acc_ref[...] += part

    @pl.when(b == nb - 1)
    def _fin():
        n_el = nb * _ED * 1024
        loss_ref[...] = jnp.clip((1.0 + _CC) * acc_ref[...] / n_el, 0.0, 5.0)


def kernel(z, emb):
    B, C, H, W = z.shape
    hw = H * W
    z3 = z.reshape(B, C, hw)
    embT = emb.T
    ehi = embT.astype(jnp.bfloat16)
    elo = (embT - ehi.astype(jnp.float32)).astype(jnp.bfloat16)

    q3, idx3, loss = pl.pallas_call(
        _vq_body,
        grid=(B,),
        in_specs=[
            pl.BlockSpec((1, C, hw), lambda b: (b, 0, 0)),
            pl.BlockSpec((_NE, _ED), lambda b: (0, 0)),
            pl.BlockSpec((_ED, _NE), lambda b: (0, 0)),
            pl.BlockSpec((_ED, _NE), lambda b: (0, 0)),
        ],
        out_specs=[
            pl.BlockSpec((1, C, hw), lambda b: (b, 0, 0)),
            pl.BlockSpec((1, 1, hw), lambda b: (b, 0, 0)),
            pl.BlockSpec((1, 1), lambda b: (0, 0)),
        ],
        out_shape=[
            jax.ShapeDtypeStruct((B, C, hw), jnp.float32),
            jax.ShapeDtypeStruct((B, 1, hw), jnp.int32),
            jax.ShapeDtypeStruct((1, 1), jnp.float32),
        ],
        scratch_shapes=[pltpu.VMEM((1, 1), jnp.float32)],
        compiler_params=pltpu.CompilerParams(
            dimension_semantics=("arbitrary",)),
    )(z3, emb, ehi, elo)

    quantized_st = q3.reshape(B, C, H, W)
    indices = idx3.reshape(B, H, W)
    return (quantized_st, indices, loss.reshape(()))


# speedup vs baseline: 2.2380x; 1.1823x over previous
import jax
import jax.numpy as jnp
from jax.experimental import pallas as pl
from jax.experimental.pallas import tpu as pltpu

_NE = 1024
_ED = 64
_CC = 0.25
_EPS = 1e-12
_KA = 72
_MG = 144


def _vq_body(z_ref, emb_ref, g_ref, q_ref, idx_ref, loss_ref,
             a_ref, f_ref, acc_ref):
    b = pl.program_id(0)
    nb = pl.num_programs(0)

    @pl.when(b == 0)
    def _prep():
        emb = emb_ref[...]
        n = jnp.clip(jnp.sqrt(jnp.sum(emb * emb, axis=1, keepdims=True)),
                     _EPS, None)
        en = emb / n
        c = jnp.sum(en * en, axis=1, keepdims=True)
        chi = c.astype(jnp.bfloat16)
        r1 = c - chi.astype(jnp.float32)
        cmid = r1.astype(jnp.bfloat16)
        clo = (r1 - cmid.astype(jnp.float32)).astype(jnp.bfloat16)
        en2 = (en + en).astype(jnp.bfloat16)
        a_ref[...] = jnp.concatenate(
            [en2, -chi, -cmid, -clo,
             jnp.zeros((_NE, _KA - _ED - 3), jnp.bfloat16)], axis=1)
        f_ref[_ED:, :] = jnp.concatenate(
            [jnp.ones((3, _NE), jnp.bfloat16),
             jnp.zeros((_KA - _ED - 3, _NE), jnp.bfloat16)], axis=0)
        acc_ref[...] = jnp.zeros_like(acc_ref)

    zb = z_ref[0]
    ssq = jnp.sum(zb * zb, axis=0, keepdims=True)
    fn = zb / jnp.clip(jnp.sqrt(ssq), _EPS, None)
    f_ref[0:_ED, :] = fn.astype(jnp.bfloat16)

    s = jax.lax.dot_general(a_ref[...], f_ref[...],
                            (((1,), (0,)), ((), ())),
                            preferred_element_type=jnp.float32)
    m = jnp.max(s, axis=0, keepdims=True)
    oh = (s == m).astype(jnp.bfloat16)

    qq = jax.lax.dot_general(g_ref[...], oh, (((1,), (0,)), ((), ())),
                             preferred_element_type=jnp.float32)
    q = qq[0:_ED] + qq[_ED:2 * _ED]
    idxf = qq[2 * _ED:2 * _ED + 1] + qq[2 * _ED + 1:2 * _ED + 2]
    cnt = qq[2 * _ED + 2:2 * _ED + 3]
    q_ref[0] = q
    idx_ref[0] = idxf.astype(jnp.int32)

    @pl.when(jnp.max(cnt) > 1.5)
    def _ties():
        jids = jax.lax.broadcasted_iota(jnp.int32, (_NE, _NE), 0)
        idx = jnp.min(jnp.where(s == m, jids, _NE), axis=0, keepdims=True)
        oh1 = (jids == idx).astype(jnp.bfloat16)
        q1 = jax.lax.dot_general(g_ref[...], oh1, (((1,), (0,)), ((), ())),
                                 preferred_element_type=jnp.float32)
        q_ref[0] = q1[0:_ED] + q1[_ED:2 * _ED]
        idx_ref[0] = idx

    diff = q_ref[0] - zb
    acc_ref[...] += jnp.sum(diff * diff).reshape(1, 1)

    @pl.when(b == nb - 1)
    def _fin():
        n_el = nb * _ED * _NE
        loss_ref[...] = jnp.clip((1.0 + _CC) * acc_ref[...] / n_el, 0.0, 5.0)


def kernel(z, emb):
    B, C, H, W = z.shape
    hw = H * W
    z3 = z.reshape(B, C, hw)

    embT = emb.T
    ehi = embT.astype(jnp.bfloat16)
    elo = (embT - ehi.astype(jnp.float32)).astype(jnp.bfloat16)
    jrow = jnp.arange(_NE, dtype=jnp.float32).reshape(1, _NE)
    jhi = jrow.astype(jnp.bfloat16)
    jlo = (jrow - jhi.astype(jnp.float32)).astype(jnp.bfloat16)
    g = jnp.concatenate(
        [ehi, elo, jhi, jlo, jnp.ones((1, _NE), jnp.bfloat16),
         jnp.zeros((_MG - 2 * _ED - 3, _NE), jnp.bfloat16)], axis=0)

    q3, idx3, loss = pl.pallas_call(
        _vq_body,
        grid=(B,),
        in_specs=[
            pl.BlockSpec((1, C, hw), lambda b: (b, 0, 0)),
            pl.BlockSpec((_NE, _ED), lambda b: (0, 0)),
            pl.BlockSpec((_MG, _NE), lambda b: (0, 0)),
        ],
        out_specs=[
            pl.BlockSpec((1, C, hw), lambda b: (b, 0, 0)),
            pl.BlockSpec((1, 1, hw), lambda b: (b, 0, 0)),
            pl.BlockSpec((1, 1), lambda b: (0, 0)),
        ],
        out_shape=[
            jax.ShapeDtypeStruct((B, C, hw), jnp.float32),
            jax.ShapeDtypeStruct((B, 1, hw), jnp.int32),
            jax.ShapeDtypeStruct((1, 1), jnp.float32),
        ],
        scratch_shapes=[
            pltpu.VMEM((_NE, _KA), jnp.bfloat16),
            pltpu.VMEM((_KA, _NE), jnp.bfloat16),
            pltpu.VMEM((1, 1), jnp.float32),
        ],
        compiler_params=pltpu.CompilerParams(
            dimension_semantics=("arbitrary",)),
    )(z3, emb, g)

    quantized_st = q3.reshape(B, C, H, W)
    indices = idx3.reshape(B, H, W)
    return (quantized_st, indices, loss.reshape(()))


# speedup vs baseline: 2.4755x; 1.1061x over previous
import jax
import jax.numpy as jnp
from jax.experimental import pallas as pl
from jax.experimental.pallas import tpu as pltpu

_NE = 1024
_ED = 64
_CC = 0.25
_EPS = 1e-12
_KA = 72
_MG = 144
_GB = 2
_NP = _GB * _NE


def _vq_body(z_ref, emb_ref, g_ref, q_ref, idx_ref, loss_ref,
             a_ref, f_ref, acc_ref):
    b = pl.program_id(0)
    nb = pl.num_programs(0)

    @pl.when(b == 0)
    def _prep():
        emb = emb_ref[...]
        n = jnp.clip(jnp.sqrt(jnp.sum(emb * emb, axis=1, keepdims=True)),
                     _EPS, None)
        en = emb / n
        c = jnp.sum(en * en, axis=1, keepdims=True)
        chi = c.astype(jnp.bfloat16)
        r1 = c - chi.astype(jnp.float32)
        cmid = r1.astype(jnp.bfloat16)
        clo = (r1 - cmid.astype(jnp.float32)).astype(jnp.bfloat16)
        en2 = (en + en).astype(jnp.bfloat16)
        a_ref[...] = jnp.concatenate(
            [en2, -chi, -cmid, -clo,
             jnp.zeros((_NE, _KA - _ED - 3), jnp.bfloat16)], axis=1)
        f_ref[_ED:, :] = jnp.concatenate(
            [jnp.ones((3, _NP), jnp.bfloat16),
             jnp.zeros((_KA - _ED - 3, _NP), jnp.bfloat16)], axis=0)
        acc_ref[...] = jnp.zeros_like(acc_ref)

    for i in range(_GB):
        zi = z_ref[i]
        ssq = jnp.sum(zi * zi, axis=0, keepdims=True)
        fni = zi / jnp.clip(jnp.sqrt(ssq), _EPS, None)
        f_ref[0:_ED, i * _NE:(i + 1) * _NE] = fni.astype(jnp.bfloat16)

    s = jax.lax.dot_general(a_ref[...], f_ref[...],
                            (((1,), (0,)), ((), ())),
                            preferred_element_type=jnp.float32)
    m = jnp.max(s, axis=0, keepdims=True)
    oh = (s == m).astype(jnp.bfloat16)

    qq = jax.lax.dot_general(g_ref[...], oh, (((1,), (0,)), ((), ())),
                             preferred_element_type=jnp.float32)
    q = qq[0:_ED] + qq[_ED:2 * _ED]
    idxf = qq[2 * _ED:2 * _ED + 1] + qq[2 * _ED + 1:2 * _ED + 2]
    cnt = qq[2 * _ED + 2:2 * _ED + 3]
    idxi = idxf.astype(jnp.int32)
    for i in range(_GB):
        q_ref[i] = q[:, i * _NE:(i + 1) * _NE]
        idx_ref[i, 0:1, :] = idxi[:, i * _NE:(i + 1) * _NE]

    @pl.when(jnp.max(cnt) > 1.5)
    def _ties():
        jids = jax.lax.broadcasted_iota(jnp.int32, (_NE, _NP), 0)
        idx = jnp.min(jnp.where(s == m, jids, _NE), axis=0, keepdims=True)
        oh1 = (jids == idx).astype(jnp.bfloat16)
        q1 = jax.lax.dot_general(g_ref[...], oh1, (((1,), (0,)), ((), ())),
                                 preferred_element_type=jnp.float32)
        qt = q1[0:_ED] + q1[_ED:2 * _ED]
        for i in range(_GB):
            q_ref[i] = qt[:, i * _NE:(i + 1) * _NE]
            idx_ref[i, 0:1, :] = idx[:, i * _NE:(i + 1) * _NE]

    part = jnp.zeros((1, 1), jnp.float32)
    for i in range(_GB):
        diff = q_ref[i] - z_ref[i]
        part = part + jnp.sum(diff * diff).reshape(1, 1)
    acc_ref[...] += part

    @pl.when(b == nb - 1)
    def _fin():
        n_el = nb * _ED * _NP
        loss_ref[...] = jnp.clip((1.0 + _CC) * acc_ref[...] / n_el, 0.0, 5.0)


def kernel(z, emb):
    B, C, H, W = z.shape
    hw = H * W
    z3 = z.reshape(B, C, hw)

    embT = emb.T
    ehi = embT.astype(jnp.bfloat16)
    elo = (embT - ehi.astype(jnp.float32)).astype(jnp.bfloat16)
    jrow = jnp.arange(_NE, dtype=jnp.float32).reshape(1, _NE)
    jhi = jrow.astype(jnp.bfloat16)
    jlo = (jrow - jhi.astype(jnp.float32)).astype(jnp.bfloat16)
    g = jnp.concatenate(
        [ehi, elo, jhi, jlo, jnp.ones((1, _NE), jnp.bfloat16),
         jnp.zeros((_MG - 2 * _ED - 3, _NE), jnp.bfloat16)], axis=0)

    q3, idx3, loss = pl.pallas_call(
        _vq_body,
        grid=(B // _GB,),
        in_specs=[
            pl.BlockSpec((_GB, C, hw), lambda b: (b, 0, 0)),
            pl.BlockSpec((_NE, _ED), lambda b: (0, 0)),
            pl.BlockSpec((_MG, _NE), lambda b: (0, 0)),
        ],
        out_specs=[
            pl.BlockSpec((_GB, C, hw), lambda b: (b, 0, 0)),
            pl.BlockSpec((_GB, 1, hw), lambda b: (b, 0, 0)),
            pl.BlockSpec((1, 1), lambda b: (0, 0)),
        ],
        out_shape=[
            jax.ShapeDtypeStruct((B, C, hw), jnp.float32),
            jax.ShapeDtypeStruct((B, 1, hw), jnp.int32),
            jax.ShapeDtypeStruct((1, 1), jnp.float32),
        ],
        scratch_shapes=[
            pltpu.VMEM((_NE, _KA), jnp.bfloat16),
            pltpu.VMEM((_KA, _NP), jnp.bfloat16),
            pltpu.VMEM((1, 1), jnp.float32),
        ],
        compiler_params=pltpu.CompilerParams(
            dimension_semantics=("arbitrary",)),
    )(z3, emb, g)

    quantized_st = q3.reshape(B, C, H, W)
    indices = idx3.reshape(B, H, W)
    return (quantized_st, indices, loss.reshape(()))


# speedup vs baseline: 2.5949x; 1.0482x over previous
import jax
import jax.numpy as jnp
from jax.experimental import pallas as pl
from jax.experimental.pallas import tpu as pltpu

_NE = 1024
_ED = 64
_CC = 0.25
_EPS = 1e-12
_KA = 72
_MG = 144
_GB = 4
_NP = _GB * _NE


def _vq_body(z_ref, emb_ref, g_ref, q_ref, idx_ref, loss_ref,
             a_ref, f_ref, acc_ref):
    b = pl.program_id(0)
    nb = pl.num_programs(0)

    @pl.when(b == 0)
    def _prep():
        emb = emb_ref[...]
        n = jnp.clip(jnp.sqrt(jnp.sum(emb * emb, axis=1, keepdims=True)),
                     _EPS, None)
        en = emb / n
        c = jnp.sum(en * en, axis=1, keepdims=True)
        chi = c.astype(jnp.bfloat16)
        r1 = c - chi.astype(jnp.float32)
        cmid = r1.astype(jnp.bfloat16)
        clo = (r1 - cmid.astype(jnp.float32)).astype(jnp.bfloat16)
        en2 = (en + en).astype(jnp.bfloat16)
        a_ref[...] = jnp.concatenate(
            [en2, -chi, -cmid, -clo,
             jnp.zeros((_NE, _KA - _ED - 3), jnp.bfloat16)], axis=1)
        f_ref[_ED:, :] = jnp.concatenate(
            [jnp.ones((3, _NP), jnp.bfloat16),
             jnp.zeros((_KA - _ED - 3, _NP), jnp.bfloat16)], axis=0)
        acc_ref[...] = jnp.zeros_like(acc_ref)

    for i in range(_GB):
        zi = z_ref[i]
        ssq = jnp.sum(zi * zi, axis=0, keepdims=True)
        fni = zi / jnp.clip(jnp.sqrt(ssq), _EPS, None)
        f_ref[0:_ED, i * _NE:(i + 1) * _NE] = fni.astype(jnp.bfloat16)

    s = jax.lax.dot_general(a_ref[...], f_ref[...],
                            (((1,), (0,)), ((), ())),
                            preferred_element_type=jnp.float32)
    m = jnp.max(s, axis=0, keepdims=True)
    oh = (s == m).astype(jnp.bfloat16)

    qq = jax.lax.dot_general(g_ref[...], oh, (((1,), (0,)), ((), ())),
                             preferred_element_type=jnp.float32)
    q = qq[0:_ED] + qq[_ED:2 * _ED]
    idxf = qq[2 * _ED:2 * _ED + 1] + qq[2 * _ED + 1:2 * _ED + 2]
    cnt = qq[2 * _ED + 2:2 * _ED + 3]
    idxi = idxf.astype(jnp.int32)
    for i in range(_GB):
        q_ref[i] = q[:, i * _NE:(i + 1) * _NE]
        idx_ref[i, 0:1, :] = idxi[:, i * _NE:(i + 1) * _NE]

    @pl.when(jnp.max(cnt) > 1.5)
    def _ties():
        jids = jax.lax.broadcasted_iota(jnp.int32, (_NE, _NP), 0)
        idx = jnp.min(jnp.where(s == m, jids, _NE), axis=0, keepdims=True)
        oh1 = (jids == idx).astype(jnp.bfloat16)
        q1 = jax.lax.dot_general(g_ref[...], oh1, (((1,), (0,)), ((), ())),
                                 preferred_element_type=jnp.float32)
        qt = q1[0:_ED] + q1[_ED:2 * _ED]
        for i in range(_GB):
            q_ref[i] = qt[:, i * _NE:(i + 1) * _NE]
            idx_ref[i, 0:1, :] = idx[:, i * _NE:(i + 1) * _NE]

    part = jnp.zeros((1, 1), jnp.float32)
    for i in range(_GB):
        diff = q_ref[i] - z_ref[i]
        part = part + jnp.sum(diff * diff).reshape(1, 1)
    acc_ref[...] += part

    @pl.when(b == nb - 1)
    def _fin():
        n_el = nb * _ED * _NP
        loss_ref[...] = jnp.clip((1.0 + _CC) * acc_ref[...] / n_el, 0.0, 5.0)


def kernel(z, emb):
    B, C, H, W = z.shape
    hw = H * W
    z3 = z.reshape(B, C, hw)

    embT = emb.T
    ehi = embT.astype(jnp.bfloat16)
    elo = (embT - ehi.astype(jnp.float32)).astype(jnp.bfloat16)
    jrow = jnp.arange(_NE, dtype=jnp.float32).reshape(1, _NE)
    jhi = jrow.astype(jnp.bfloat16)
    jlo = (jrow - jhi.astype(jnp.float32)).astype(jnp.bfloat16)
    g = jnp.concatenate(
        [ehi, elo, jhi, jlo, jnp.ones((1, _NE), jnp.bfloat16),
         jnp.zeros((_MG - 2 * _ED - 3, _NE), jnp.bfloat16)], axis=0)

    q3, idx3, loss = pl.pallas_call(
        _vq_body,
        grid=(B // _GB,),
        in_specs=[
            pl.BlockSpec((_GB, C, hw), lambda b: (b, 0, 0)),
            pl.BlockSpec((_NE, _ED), lambda b: (0, 0)),
            pl.BlockSpec((_MG, _NE), lambda b: (0, 0)),
        ],
        out_specs=[
            pl.BlockSpec((_GB, C, hw), lambda b: (b, 0, 0)),
            pl.BlockSpec((_GB, 1, hw), lambda b: (b, 0, 0)),
            pl.BlockSpec((1, 1), lambda b: (0, 0)),
        ],
        out_shape=[
            jax.ShapeDtypeStruct((B, C, hw), jnp.float32),
            jax.ShapeDtypeStruct((B, 1, hw), jnp.int32),
            jax.ShapeDtypeStruct((1, 1), jnp.float32),
        ],
        scratch_shapes=[
            pltpu.VMEM((_NE, _KA), jnp.bfloat16),
            pltpu.VMEM((_KA, _NP), jnp.bfloat16),
            pltpu.VMEM((1, 1), jnp.float32),
        ],
        compiler_params=pltpu.CompilerParams(
            dimension_semantics=("arbitrary",)),
    )(z3, emb, g)

    quantized_st = q3.reshape(B, C, H, W)
    indices = idx3.reshape(B, H, W)
    return (quantized_st, indices, loss.reshape(()))
